# baseline (speedup 1.0000x reference)
import functools

import jax
import jax.numpy as jnp
from jax import lax
from jax.experimental import pallas as pl
from jax.experimental.pallas import tpu as pltpu
from jax.experimental.pallas import tpu_sc as plsc

N = 10000
E = 320000
C = 128
NC = 2
NS = 16
NW = NC * NS
EPW = E // NW
EPS = E // NS
SUPER = 4000
SUB = 80
NPAIR = SUPER // (2 * SUB)
CQ = C // 4
CA = 3 * CQ


def _tc_a(x, gat_W, att_src, att_dst):
    def body(x_ref, w_ref, s_ref, d_ref, asrc_ref, adst_ref, gself_ref):
        vs = jnp.sum(w_ref[...] * s_ref[...][None, :], axis=1)
        vd = jnp.sum(w_ref[...] * d_ref[...][None, :], axis=1)
        a_src = jnp.sum(x_ref[...] * vs[None, :], axis=1)
        a_dst = jnp.sum(x_ref[...] * vd[None, :], axis=1)
        e = a_src + a_dst
        e = jnp.where(e >= 0, e, 0.2 * e)
        asrc_ref[...] = a_src
        adst_ref[...] = a_dst
        gself_ref[...] = jnp.exp(e)

    return pl.pallas_call(
        body,
        out_shape=(jax.ShapeDtypeStruct((N,), jnp.float32),
                   jax.ShapeDtypeStruct((N,), jnp.float32),
                   jax.ShapeDtypeStruct((N,), jnp.float32)),
    )(x, gat_W, att_src, att_dst)


def _sc_b(row, col, asrc, adst):
    mesh = plsc.VectorSubcoreMesh(core_axis_name="c", subcore_axis_name="s")

    @functools.partial(
        pl.kernel,
        out_type=(jax.ShapeDtypeStruct((NW * N,), jnp.float32),
                  jax.ShapeDtypeStruct((NW * N,), jnp.float32),
                  jax.ShapeDtypeStruct((E,), jnp.float32)),
        mesh=mesh,
        compiler_params=pltpu.CompilerParams(needs_layout_passes=False),
        scratch_types=[
            pltpu.VMEM((N,), jnp.float32),
            pltpu.VMEM((N,), jnp.float32),
            pltpu.VMEM((N,), jnp.float32),
            pltpu.VMEM((N,), jnp.float32),
            pltpu.VMEM((EPW,), jnp.int32),
            pltpu.VMEM((EPW,), jnp.int32),
            pltpu.VMEM((EPW,), jnp.float32),
        ],
    )
    def k(row_h, col_h, asrc_h, adst_h, cnt_h, d_h, g_h,
          asrc_v, adst_v, cacc, dacc, rowb, colb, gb):
        cid = lax.axis_index("c")
        sid = lax.axis_index("s")
        wid = sid * NC + cid
        base = wid * EPW
        pltpu.sync_copy(asrc_h, asrc_v)
        pltpu.sync_copy(adst_h, adst_v)
        pltpu.sync_copy(row_h.at[pl.ds(base, EPW)], rowb)
        pltpu.sync_copy(col_h.at[pl.ds(base, EPW)], colb)

        zeros = jnp.zeros((16,), jnp.float32)

        def zbody(i, carry):
            cacc[pl.ds(i * 16, 16)] = zeros
            dacc[pl.ds(i * 16, 16)] = zeros
            return carry

        lax.fori_loop(0, N // 16, zbody, 0)

        ones = jnp.ones((16,), jnp.float32)

        def ebody(i, carry):
            sl = pl.ds(i * 16, 16)
            r = rowb[sl]
            c = colb[sl]
            av = plsc.load_gather(asrc_v, [r]) + plsc.load_gather(adst_v, [c])
            e = jnp.where(av >= 0, av, 0.2 * av)
            g = jnp.exp(e)
            gb[sl] = g
            plsc.addupdate_scatter(dacc, [c], g)
            plsc.addupdate_scatter(cacc, [c], ones)
            return carry

        lax.fori_loop(0, EPW // 16, ebody, 0)

        pltpu.sync_copy(cacc, cnt_h.at[pl.ds(wid * N, N)])
        pltpu.sync_copy(dacc, d_h.at[pl.ds(wid * N, N)])
        pltpu.sync_copy(gb, g_h.at[pl.ds(base, EPW)])

    return k(row, col, asrc, adst)


def _tc_c(cnt_part, d_part, gself):
    def body(cp_ref, dp_ref, gs_ref, dis_ref, invc_ref, dinv_ref):
        cnt = jnp.sum(cp_ref[...], axis=0)
        d = jnp.sum(dp_ref[...], axis=0) + gs_ref[...]
        dis_ref[...] = lax.rsqrt(cnt + 1.0)
        invc_ref[...] = 1.0 / jnp.maximum(cnt, 1.0)
        dinv_ref[...] = 1.0 / d

    return pl.pallas_call(
        body,
        out_shape=(jax.ShapeDtypeStruct((N,), jnp.float32),
                   jax.ShapeDtypeStruct((N,), jnp.float32),
                   jax.ShapeDtypeStruct((N,), jnp.float32)),
    )(cnt_part, d_part, gself)


def _sc_d(row, col, g, dis, xquart):
    mesh = plsc.VectorSubcoreMesh(core_axis_name="c", subcore_axis_name="s")
    nsuper = EPS // SUPER

    @functools.partial(
        pl.kernel,
        out_type=jax.ShapeDtypeStruct((4 * N, CA), jnp.float32),
        mesh=mesh,
        compiler_params=pltpu.CompilerParams(needs_layout_passes=False,
                                             use_tc_tiling_on_sc=False),
        scratch_types=[
            pltpu.VMEM((N,), jnp.float32),
            pltpu.VMEM((SUPER,), jnp.int32),
            pltpu.VMEM((SUPER,), jnp.int32),
            pltpu.VMEM((SUPER,), jnp.float32),
            pltpu.VMEM((SUPER,), jnp.float32),
            pltpu.VMEM((SUB,), jnp.int32),
            pltpu.VMEM((SUB,), jnp.int32),
            pltpu.VMEM((SUB,), jnp.int32),
            pltpu.VMEM((SUB,), jnp.int32),
            pltpu.VMEM((SUB, CQ), jnp.float32),
            pltpu.VMEM((SUB, CQ), jnp.float32),
            pltpu.VMEM((SUB, CA), jnp.float32),
            pltpu.VMEM((SUB, CA), jnp.float32),
            pltpu.VMEM((16, CA), jnp.float32),
            pltpu.VMEM_SHARED((N + 16, CA), jnp.float32),
            pltpu.SemaphoreType.DMA,
            pltpu.SemaphoreType.DMA,
            pltpu.SemaphoreType.DMA,
            pltpu.SemaphoreType.DMA,
        ],
    )
    def k(row_h, col_h, g_h, dis_h, xs_h, aall_h,
          dis_v, rowB, colB, gB, s1B, idx0, col0, idx1, col1, xg0, xg1,
          st0, st1b, zbuf, acc, semg0, semg1, sems0, sems1):
        cid = lax.axis_index("c")
        sid = lax.axis_index("s")
        pltpu.sync_copy(dis_h, dis_v)

        zeros = jnp.zeros((16,), jnp.float32)
        izeros = jnp.zeros((16,), jnp.int32)

        def zb(i, carry):
            r = i // (CA // 16)
            q = i % (CA // 16)
            zbuf[r, pl.ds(q * 16, 16)] = zeros
            return carry

        lax.fori_loop(0, 16 * (CA // 16), zb, 0)

        rowbase = sid * 624
        nchunk = jnp.where(sid == NS - 1, 40, 39)
        ebase = sid * EPS

        def zacc(i, carry):
            sl = pl.ds(rowbase + i * 16, 16)
            pltpu.sync_copy(zbuf, acc.at[sl, :])
            return carry

        trash = jnp.full((16,), N, jnp.int32)

        def zcol(i, carry):
            col0[pl.ds(i * 16, 16)] = trash
            col1[pl.ds(i * 16, 16)] = trash
            return carry

        for qpass in range(2):
            lax.fori_loop(0, SUB // 16, zcol, 0)
            pltpu.async_copy(st0, acc.at[col0], sems0, add=True)
            pltpu.async_copy(st1b, acc.at[col1], sems1, add=True)
            lax.fori_loop(0, nchunk, zacc, 0)
            plsc.subcore_barrier()
            quart = 2 * cid + qpass

            def build(offe, idxr):
                def ib(i, c3):
                    sl16 = pl.ds(i * 16, 16)
                    idxr[sl16] = rowB[pl.ds(offe + i * 16, 16)] + quart * N
                    return c3

                lax.fori_loop(0, SUB // 16, ib, 0)

            def build_col(offe, colr):
                def ib(i, c3):
                    sl16 = pl.ds(i * 16, 16)
                    colr[sl16] = colB[pl.ds(offe + i * 16, 16)]
                    return c3

                lax.fori_loop(0, SUB // 16, ib, 0)

            def compute(offe, xgr, stw):
                def grp(t, c2):
                    base = offe + t * 16
                    s1v = s1B[pl.ds(base, 16)]
                    gv = gB[pl.ds(base, 16)]
                    for jj in range(16):
                        j = t * 16 + jj
                        s1 = s1v[jj]
                        gj = gv[jj]
                        for q in range(CQ // 16):
                            sl = pl.ds(q * 16, 16)
                            xv = xgr[j, sl]
                            stw[j, pl.ds(q * 16, 16)] = s1 * xv
                            stw[j, pl.ds(CQ + q * 16, 16)] = xv
                            stw[j, pl.ds(2 * CQ + q * 16, 16)] = gj * xv
                    return c2

                lax.fori_loop(0, SUB // 16, grp, 0)

            def super_body(u, carry):
                sb = ebase + u * SUPER
                pltpu.sync_copy(row_h.at[pl.ds(sb, SUPER)], rowB)
                pltpu.sync_copy(col_h.at[pl.ds(sb, SUPER)], colB)
                pltpu.sync_copy(g_h.at[pl.ds(sb, SUPER)], gB)

                def s1body(i, c2):
                    sl = pl.ds(i * 16, 16)
                    s1B[sl] = plsc.load_gather(dis_v, [rowB[sl]])
                    return c2

                lax.fori_loop(0, SUPER // 16, s1body, 0)

                build(0, idx0)
                pltpu.async_copy(xs_h.at[idx0], xg0, semg0)

                def pair(w, c2):
                    offa = 2 * w * SUB
                    offb = offa + SUB
                    pltpu.make_async_copy(st1b, acc.at[col1], sems1).wait()
                    build(offb, idx1)
                    pltpu.async_copy(xs_h.at[idx1], xg1, semg1)
                    pltpu.make_async_copy(xs_h.at[idx0], xg0, semg0).wait()
                    pltpu.make_async_copy(st0, acc.at[col0], sems0).wait()
                    compute(offa, xg0, st0)
                    build_col(offa, col0)
                    pltpu.async_copy(st0, acc.at[col0], sems0, add=True)

                    @pl.when(w < NPAIR - 1)
                    def _prefetch():
                        build(offa + 2 * SUB, idx0)
                        pltpu.async_copy(xs_h.at[idx0], xg0, semg0)

                    pltpu.make_async_copy(xs_h.at[idx1], xg1, semg1).wait()
                    compute(offb, xg1, st1b)
                    build_col(offb, col1)
                    pltpu.async_copy(st1b, acc.at[col1], sems1, add=True)
                    return c2

                lax.fori_loop(0, NPAIR, pair, 0)
                return carry

            lax.fori_loop(0, nsuper, super_body, 0)
            pltpu.make_async_copy(st0, acc.at[col0], sems0).wait()
            pltpu.make_async_copy(st1b, acc.at[col1], sems1).wait()
            plsc.subcore_barrier()

            outbase = quart * N + rowbase

            def cp(i, carry):
                src = pl.ds(rowbase + i * 16, 16)
                dst = pl.ds(outbase + i * 16, 16)
                pltpu.sync_copy(acc.at[src, :], aall_h.at[dst, :])
                return carry

            lax.fori_loop(0, nchunk, cp, 0)
            if qpass == 0:
                plsc.subcore_barrier()

    return k(row, col, g, dis, xquart)


def _tc_e(x, a1, a2, a3, dis, invc, dinv, gself, weights,
          gcn_W, sage_Wl, sage_Wr, gat_W, gcn_b, sage_b, gat_b):
    RB = 1000
    grid = N // RB

    def body(w_ref, x_ref, a1_ref, a2_ref, a3_ref, dis_ref, ic_ref, di_ref,
             gs_ref, gw_ref, wl_ref, wr_ref, gaw_ref, gb_ref, sb_ref, ab_ref,
             o_ref):
        w0 = w_ref[0]
        w1 = w_ref[1]
        w2 = w_ref[2]
        w3 = w_ref[3]
        xb = x_ref[...]
        dis = dis_ref[...]
        p1 = dis * (a1_ref[...] + dis * xb)
        p2 = a2_ref[...] * ic_ref[...]
        p3 = (a3_ref[...] + gs_ref[...] * xb) * di_ref[...]
        out = jnp.dot(w0 * p1, gw_ref[...], preferred_element_type=jnp.float32)
        out = out + jnp.dot(w1 * p2, wl_ref[...],
                            preferred_element_type=jnp.float32)
        out = out + jnp.dot(w1 * xb, wr_ref[...],
                            preferred_element_type=jnp.float32)
        out = out + jnp.dot(w2 * p3, gaw_ref[...],
                            preferred_element_type=jnp.float32)
        bias = w0 * gb_ref[...] + w1 * sb_ref[...] + w2 * ab_ref[...]
        o_ref[...] = out + w3 * xb + bias[None, :]

    mat = pl.BlockSpec((RB, C), lambda i: (i, 0))
    vec = pl.BlockSpec((RB, 1), lambda i: (i, 0))
    wmat = pl.BlockSpec((C, C), lambda i: (0, 0))
    wvec = pl.BlockSpec((C,), lambda i: (0,))
    return pl.pallas_call(
        body,
        grid=(grid,),
        in_specs=[pl.BlockSpec(memory_space=pltpu.SMEM),
                  mat, mat, mat, mat, vec, vec, vec, vec,
                  wmat, wmat, wmat, wmat, wvec, wvec, wvec],
        out_specs=mat,
        out_shape=jax.ShapeDtypeStruct((N, C), jnp.float32),
    )(weights, x, a1, a2, a3, dis[:, None], invc[:, None], dinv[:, None],
      gself[:, None], gcn_W, sage_Wl, sage_Wr, gat_W, gcn_b, sage_b, gat_b)


def kernel(x, edge_index, weights, gcn_W, gcn_b, sage_Wl, sage_Wr, sage_b,
           gat_W, gat_att_src, gat_att_dst, gat_b):
    row = edge_index[0]
    col = edge_index[1]
    asrc, adst, gself = _tc_a(x, gat_W, gat_att_src, gat_att_dst)
    cnt_p, d_p, g = _sc_b(row, col, asrc, adst)
    dis, invc, dinv = _tc_c(cnt_p.reshape(NW, N), d_p.reshape(NW, N), gself)
    xquart = jnp.concatenate([x[:, q * CQ:(q + 1) * CQ] for q in range(4)],
                             axis=0)
    aall = _sc_d(row, col, g, dis, xquart)
    a1 = jnp.concatenate([aall[q * N:(q + 1) * N, 0:CQ] for q in range(4)],
                         axis=1)
    a2 = jnp.concatenate([aall[q * N:(q + 1) * N, CQ:2 * CQ]
                          for q in range(4)], axis=1)
    a3 = jnp.concatenate([aall[q * N:(q + 1) * N, 2 * CQ:CA]
                          for q in range(4)], axis=1)
    return _tc_e(x, a1, a2, a3, dis, invc, dinv, gself, weights,
                 gcn_W, sage_Wl, sage_Wr, gat_W, gcn_b, sage_b, gat_b)


# speedup vs baseline: 1.2601x; 1.2601x over previous
import functools

import jax
import jax.numpy as jnp
from jax import lax
from jax.experimental import pallas as pl
from jax.experimental.pallas import tpu as pltpu
from jax.experimental.pallas import tpu_sc as plsc

N = 10000
E = 320000
C = 128
NC = 2
NS = 16
NW = NC * NS
EPW = E // NW
EPS = E // NS
SUPER = 4000
SUB = 80
NPAIR = SUPER // (2 * SUB)
CQ = C // 4
CA = 3 * CQ


def _tc_a(x, gat_W, att_src, att_dst):
    def body(x_ref, w_ref, s_ref, d_ref, asrc_ref, adst_ref, gself_ref):
        vs = jnp.sum(w_ref[...] * s_ref[...][None, :], axis=1)
        vd = jnp.sum(w_ref[...] * d_ref[...][None, :], axis=1)
        a_src = jnp.sum(x_ref[...] * vs[None, :], axis=1)
        a_dst = jnp.sum(x_ref[...] * vd[None, :], axis=1)
        e = a_src + a_dst
        e = jnp.where(e >= 0, e, 0.2 * e)
        asrc_ref[...] = a_src
        adst_ref[...] = a_dst
        gself_ref[...] = jnp.exp(e)

    return pl.pallas_call(
        body,
        out_shape=(jax.ShapeDtypeStruct((N,), jnp.float32),
                   jax.ShapeDtypeStruct((N,), jnp.float32),
                   jax.ShapeDtypeStruct((N,), jnp.float32)),
    )(x, gat_W, att_src, att_dst)


def _sc_b(row, col, asrc, adst):
    mesh = plsc.VectorSubcoreMesh(core_axis_name="c", subcore_axis_name="s")

    @functools.partial(
        pl.kernel,
        out_type=(jax.ShapeDtypeStruct((NW * N,), jnp.float32),
                  jax.ShapeDtypeStruct((NW * N,), jnp.float32),
                  jax.ShapeDtypeStruct((E,), jnp.float32)),
        mesh=mesh,
        compiler_params=pltpu.CompilerParams(needs_layout_passes=False),
        scratch_types=[
            pltpu.VMEM((N,), jnp.float32),
            pltpu.VMEM((N,), jnp.float32),
            pltpu.VMEM((N,), jnp.float32),
            pltpu.VMEM((N,), jnp.float32),
            pltpu.VMEM((EPW,), jnp.int32),
            pltpu.VMEM((EPW,), jnp.int32),
            pltpu.VMEM((EPW,), jnp.float32),
        ],
    )
    def k(row_h, col_h, asrc_h, adst_h, cnt_h, d_h, g_h,
          asrc_v, adst_v, cacc, dacc, rowb, colb, gb):
        cid = lax.axis_index("c")
        sid = lax.axis_index("s")
        wid = sid * NC + cid
        base = wid * EPW
        pltpu.sync_copy(asrc_h, asrc_v)
        pltpu.sync_copy(adst_h, adst_v)
        pltpu.sync_copy(row_h.at[pl.ds(base, EPW)], rowb)
        pltpu.sync_copy(col_h.at[pl.ds(base, EPW)], colb)

        zeros = jnp.zeros((16,), jnp.float32)

        def zbody(i, carry):
            cacc[pl.ds(i * 16, 16)] = zeros
            dacc[pl.ds(i * 16, 16)] = zeros
            return carry

        lax.fori_loop(0, N // 16, zbody, 0)

        ones = jnp.ones((16,), jnp.float32)

        def ebody(i, carry):
            sl = pl.ds(i * 16, 16)
            r = rowb[sl]
            c = colb[sl]
            av = plsc.load_gather(asrc_v, [r]) + plsc.load_gather(adst_v, [c])
            e = jnp.where(av >= 0, av, 0.2 * av)
            g = jnp.exp(e)
            gb[sl] = g
            plsc.addupdate_scatter(dacc, [c], g)
            plsc.addupdate_scatter(cacc, [c], ones)
            return carry

        lax.fori_loop(0, EPW // 16, ebody, 0)

        pltpu.sync_copy(cacc, cnt_h.at[pl.ds(wid * N, N)])
        pltpu.sync_copy(dacc, d_h.at[pl.ds(wid * N, N)])
        pltpu.sync_copy(gb, g_h.at[pl.ds(base, EPW)])

    return k(row, col, asrc, adst)


def _tc_c(cnt_part, d_part, gself):
    def body(cp_ref, dp_ref, gs_ref, dis_ref, invc_ref, dinv_ref):
        cnt = jnp.sum(cp_ref[...], axis=0)
        d = jnp.sum(dp_ref[...], axis=0) + gs_ref[...]
        dis_ref[...] = lax.rsqrt(cnt + 1.0)
        invc_ref[...] = 1.0 / jnp.maximum(cnt, 1.0)
        dinv_ref[...] = 1.0 / d

    return pl.pallas_call(
        body,
        out_shape=(jax.ShapeDtypeStruct((N,), jnp.float32),
                   jax.ShapeDtypeStruct((N,), jnp.float32),
                   jax.ShapeDtypeStruct((N,), jnp.float32)),
    )(cnt_part, d_part, gself)


def _sc_d(row, col, g, dis, xquart):
    mesh = plsc.VectorSubcoreMesh(core_axis_name="c", subcore_axis_name="s")
    nsuper = EPS // SUPER

    @functools.partial(
        pl.kernel,
        out_type=jax.ShapeDtypeStruct((4 * N, CA), jnp.float32),
        mesh=mesh,
        compiler_params=pltpu.CompilerParams(needs_layout_passes=False,
                                             use_tc_tiling_on_sc=False),
        scratch_types=[
            pltpu.VMEM((N,), jnp.float32),
            pltpu.VMEM((SUPER,), jnp.int32),
            pltpu.VMEM((SUPER,), jnp.int32),
            pltpu.VMEM((SUPER,), jnp.float32),
            pltpu.VMEM((SUPER,), jnp.float32),
            pltpu.VMEM((SUB,), jnp.int32),
            pltpu.VMEM((SUB,), jnp.int32),
            pltpu.VMEM((SUB,), jnp.int32),
            pltpu.VMEM((SUB,), jnp.int32),
            pltpu.VMEM((SUB, CQ), jnp.float32),
            pltpu.VMEM((SUB, CQ), jnp.float32),
            pltpu.VMEM((SUB, CA), jnp.float32),
            pltpu.VMEM((SUB, CA), jnp.float32),
            pltpu.VMEM((16, CA), jnp.float32),
            pltpu.VMEM_SHARED((N + 16, CA), jnp.float32),
            pltpu.SemaphoreType.DMA,
            pltpu.SemaphoreType.DMA,
            pltpu.SemaphoreType.DMA,
            pltpu.SemaphoreType.DMA,
        ],
    )
    def k(row_h, col_h, g_h, dis_h, xs_h, aall_h,
          dis_v, rowB, colB, gB, s1B, idx0, col0, idx1, col1, xg0, xg1,
          st0, st1b, zbuf, acc, semg0, semg1, sems0, sems1):
        cid = lax.axis_index("c")
        sid = lax.axis_index("s")
        pltpu.sync_copy(dis_h, dis_v)

        zeros = jnp.zeros((16,), jnp.float32)
        izeros = jnp.zeros((16,), jnp.int32)

        def zb(i, carry):
            r = i // (CA // 16)
            q = i % (CA // 16)
            zbuf[r, pl.ds(q * 16, 16)] = zeros
            return carry

        lax.fori_loop(0, 16 * (CA // 16), zb, 0)

        rowbase = sid * 624
        nchunk = jnp.where(sid == NS - 1, 40, 39)
        ebase = sid * EPS

        def zacc(i, carry):
            sl = pl.ds(rowbase + i * 16, 16)
            pltpu.sync_copy(zbuf, acc.at[sl, :])
            return carry

        trash = jnp.full((16,), N, jnp.int32)

        def zcol(i, carry):
            col0[pl.ds(i * 16, 16)] = trash
            col1[pl.ds(i * 16, 16)] = trash
            return carry

        for qpass in range(2):
            lax.fori_loop(0, SUB // 16, zcol, 0)
            pltpu.async_copy(st0, acc.at[col0], sems0, add=True)
            pltpu.async_copy(st1b, acc.at[col1], sems1, add=True)
            lax.fori_loop(0, nchunk, zacc, 0)
            plsc.subcore_barrier()
            quart = 2 * cid + qpass

            def build(offe, idxr):
                def ib(i, c3):
                    sl16 = pl.ds(i * 16, 16)
                    idxr[sl16] = rowB[pl.ds(offe + i * 16, 16)] + quart * N
                    return c3

                lax.fori_loop(0, SUB // 16, ib, 0)

            def build_col(offe, colr):
                def ib(i, c3):
                    sl16 = pl.ds(i * 16, 16)
                    colr[sl16] = colB[pl.ds(offe + i * 16, 16)]
                    return c3

                lax.fori_loop(0, SUB // 16, ib, 0)

            def compute(offe, xgr, stw):
                def grp(t, c2):
                    base = offe + t * 16
                    s1v = s1B[pl.ds(base, 16)]
                    gv = gB[pl.ds(base, 16)]
                    for jj in range(16):
                        j = t * 16 + jj
                        s1 = s1v[jj]
                        gj = gv[jj]
                        for q in range(CQ // 16):
                            sl = pl.ds(q * 16, 16)
                            xv = xgr[j, sl]
                            stw[j, pl.ds(q * 16, 16)] = s1 * xv
                            stw[j, pl.ds(CQ + q * 16, 16)] = xv
                            stw[j, pl.ds(2 * CQ + q * 16, 16)] = gj * xv
                    return c2

                lax.fori_loop(0, SUB // 16, grp, 0, unroll=SUB // 16)

            def super_body(u, carry):
                sb = ebase + u * SUPER
                pltpu.sync_copy(row_h.at[pl.ds(sb, SUPER)], rowB)
                pltpu.sync_copy(col_h.at[pl.ds(sb, SUPER)], colB)
                pltpu.sync_copy(g_h.at[pl.ds(sb, SUPER)], gB)

                def s1body(i, c2):
                    sl = pl.ds(i * 16, 16)
                    s1B[sl] = plsc.load_gather(dis_v, [rowB[sl]])
                    return c2

                lax.fori_loop(0, SUPER // 16, s1body, 0)

                build(0, idx0)
                pltpu.async_copy(xs_h.at[idx0], xg0, semg0)

                def pair(w, c2):
                    offa = 2 * w * SUB
                    offb = offa + SUB
                    pltpu.make_async_copy(st1b, acc.at[col1], sems1).wait()
                    build(offb, idx1)
                    pltpu.async_copy(xs_h.at[idx1], xg1, semg1)
                    pltpu.make_async_copy(xs_h.at[idx0], xg0, semg0).wait()
                    pltpu.make_async_copy(st0, acc.at[col0], sems0).wait()
                    compute(offa, xg0, st0)
                    build_col(offa, col0)
                    pltpu.async_copy(st0, acc.at[col0], sems0, add=True)

                    @pl.when(w < NPAIR - 1)
                    def _prefetch():
                        build(offa + 2 * SUB, idx0)
                        pltpu.async_copy(xs_h.at[idx0], xg0, semg0)

                    pltpu.make_async_copy(xs_h.at[idx1], xg1, semg1).wait()
                    compute(offb, xg1, st1b)
                    build_col(offb, col1)
                    pltpu.async_copy(st1b, acc.at[col1], sems1, add=True)
                    return c2

                lax.fori_loop(0, NPAIR, pair, 0)
                return carry

            lax.fori_loop(0, nsuper, super_body, 0)
            pltpu.make_async_copy(st0, acc.at[col0], sems0).wait()
            pltpu.make_async_copy(st1b, acc.at[col1], sems1).wait()
            plsc.subcore_barrier()

            outbase = quart * N + rowbase

            def cp(i, carry):
                src = pl.ds(rowbase + i * 16, 16)
                dst = pl.ds(outbase + i * 16, 16)
                pltpu.sync_copy(acc.at[src, :], aall_h.at[dst, :])
                return carry

            lax.fori_loop(0, nchunk, cp, 0)
            if qpass == 0:
                plsc.subcore_barrier()

    return k(row, col, g, dis, xquart)


def _tc_e(x, a1, a2, a3, dis, invc, dinv, gself, weights,
          gcn_W, sage_Wl, sage_Wr, gat_W, gcn_b, sage_b, gat_b):
    RB = 1000
    grid = N // RB

    def body(w_ref, x_ref, a1_ref, a2_ref, a3_ref, dis_ref, ic_ref, di_ref,
             gs_ref, gw_ref, wl_ref, wr_ref, gaw_ref, gb_ref, sb_ref, ab_ref,
             o_ref):
        w0 = w_ref[0]
        w1 = w_ref[1]
        w2 = w_ref[2]
        w3 = w_ref[3]
        xb = x_ref[...]
        dis = dis_ref[...]
        p1 = dis * (a1_ref[...] + dis * xb)
        p2 = a2_ref[...] * ic_ref[...]
        p3 = (a3_ref[...] + gs_ref[...] * xb) * di_ref[...]
        out = jnp.dot(w0 * p1, gw_ref[...], preferred_element_type=jnp.float32)
        out = out + jnp.dot(w1 * p2, wl_ref[...],
                            preferred_element_type=jnp.float32)
        out = out + jnp.dot(w1 * xb, wr_ref[...],
                            preferred_element_type=jnp.float32)
        out = out + jnp.dot(w2 * p3, gaw_ref[...],
                            preferred_element_type=jnp.float32)
        bias = w0 * gb_ref[...] + w1 * sb_ref[...] + w2 * ab_ref[...]
        o_ref[...] = out + w3 * xb + bias[None, :]

    mat = pl.BlockSpec((RB, C), lambda i: (i, 0))
    vec = pl.BlockSpec((RB, 1), lambda i: (i, 0))
    wmat = pl.BlockSpec((C, C), lambda i: (0, 0))
    wvec = pl.BlockSpec((C,), lambda i: (0,))
    return pl.pallas_call(
        body,
        grid=(grid,),
        in_specs=[pl.BlockSpec(memory_space=pltpu.SMEM),
                  mat, mat, mat, mat, vec, vec, vec, vec,
                  wmat, wmat, wmat, wmat, wvec, wvec, wvec],
        out_specs=mat,
        out_shape=jax.ShapeDtypeStruct((N, C), jnp.float32),
    )(weights, x, a1, a2, a3, dis[:, None], invc[:, None], dinv[:, None],
      gself[:, None], gcn_W, sage_Wl, sage_Wr, gat_W, gcn_b, sage_b, gat_b)


def kernel(x, edge_index, weights, gcn_W, gcn_b, sage_Wl, sage_Wr, sage_b,
           gat_W, gat_att_src, gat_att_dst, gat_b):
    row = edge_index[0]
    col = edge_index[1]
    asrc, adst, gself = _tc_a(x, gat_W, gat_att_src, gat_att_dst)
    cnt_p, d_p, g = _sc_b(row, col, asrc, adst)
    dis, invc, dinv = _tc_c(cnt_p.reshape(NW, N), d_p.reshape(NW, N), gself)
    xquart = jnp.concatenate([x[:, q * CQ:(q + 1) * CQ] for q in range(4)],
                             axis=0)
    aall = _sc_d(row, col, g, dis, xquart)
    a1 = jnp.concatenate([aall[q * N:(q + 1) * N, 0:CQ] for q in range(4)],
                         axis=1)
    a2 = jnp.concatenate([aall[q * N:(q + 1) * N, CQ:2 * CQ]
                          for q in range(4)], axis=1)
    a3 = jnp.concatenate([aall[q * N:(q + 1) * N, 2 * CQ:CA]
                          for q in range(4)], axis=1)
    return _tc_e(x, a1, a2, a3, dis, invc, dinv, gself, weights,
                 gcn_W, sage_Wl, sage_Wr, gat_W, gcn_b, sage_b, gat_b)
